# Initial kernel scaffold; baseline (speedup 1.0000x reference)
import jax
import jax.numpy as jnp
from jax.experimental import pallas as pl


def kernel(x, edge_index, edge_attr, batch_idx, W_emb, b_emb, rgcn_w, rgcn_root, rgcn_b, mf_w, mf_b, mf_root, W1, b1, W2, b2):
    raise NotImplementedError("write your pallas kernel here")


# speedup vs baseline: 2.1388x; 2.1388x over previous
import functools

import jax
import jax.numpy as jnp
from jax import lax
from jax.experimental import pallas as pl
from jax.experimental.pallas import tpu as pltpu
from jax.experimental.pallas import tpu_sc as plsc

N = 10000
E = 320000
R = 16
H = 256
HH = 128
G = 64
ND = 11
XR = 17 * N
NPAD = 10240
TS = NPAD // 16

C = 128
NCH = E // C
NCHT = 156

@functools.cache
def _mesh():
    return plsc.VectorSubcoreMesh(core_axis_name="c", subcore_axis_name="s")


def _f32(*shape):
    return jax.ShapeDtypeStruct(shape, jnp.float32)


def _i32(*shape):
    return jax.ShapeDtypeStruct(shape, jnp.int32)


def _prep_body(ea_ref, src_ref, dst_ref, et_ref, seg_ref, gix2_ref, src2_ref):
    ea = ea_ref[...]
    m = jnp.max(ea, axis=0, keepdims=True)
    it = lax.broadcasted_iota(jnp.int32, ea.shape, 0)
    et = jnp.min(jnp.where(ea == m, it, R), axis=0)
    src = src_ref[...][0, 0]
    dst = dst_ref[...][0, 0]
    et_ref[...] = et[None, None]
    seg_ref[...] = (dst * R + et)[None, None]
    gix = et * N + src
    gix2_ref[...] = jnp.stack([gix, gix + XR], axis=0)[:, None, None]
    src2_ref[...] = jnp.stack([src, src + N], axis=0)[:, None, None]


def _prep(edge_attr_t, srcm, dstm):
    cb = 16000
    return pl.pallas_call(
        _prep_body,
        grid=(20,),
        in_specs=[
            pl.BlockSpec((16, cb), lambda i: (0, i)),
            pl.BlockSpec((1, 1, cb), lambda i: (i, 0, 0)),
            pl.BlockSpec((1, 1, cb), lambda i: (i, 0, 0)),
        ],
        out_specs=[
            pl.BlockSpec((1, 1, cb), lambda i: (i, 0, 0)),
            pl.BlockSpec((1, 1, cb), lambda i: (i, 0, 0)),
            pl.BlockSpec((2, 1, 1, cb), lambda i: (0, i, 0, 0)),
            pl.BlockSpec((2, 1, 1, cb), lambda i: (0, i, 0, 0)),
        ],
        out_shape=[_i32(20, 1, cb), _i32(20, 1, cb), _i32(2, 20, 1, cb),
                   _i32(2, 20, 1, cb)],
    )(edge_attr_t, srcm, dstm)


def _emb_body(x_ref, w_ref, b_ref, o_ref):
    o_ref[...] = jnp.dot(x_ref[...], w_ref[...],
                         preferred_element_type=jnp.float32) + b_ref[...]


def _emb(x, w, b):
    return pl.pallas_call(
        _emb_body,
        grid=(25,),
        in_specs=[
            pl.BlockSpec((400, 128), lambda i: (i, 0)),
            pl.BlockSpec((128, 256), lambda i: (0, 0)),
            pl.BlockSpec((1, 256), lambda i: (0, 0)),
        ],
        out_specs=pl.BlockSpec((400, 256), lambda i: (i, 0)),
        out_shape=_f32(N, 256),
    )(x, w, b)


def _counts_body(et_hbm, dst_hbm, z128_hbm, out_hbm, etv, dstv, oh, cnt_sh):
    c = lax.axis_index("c")
    s = lax.axis_index("s")
    t = c * 16 + s
    pltpu.sync_copy(z128_hbm, cnt_sh.at[pl.ds(s * TS, TS)])
    pltpu.sync_copy(z128_hbm.at[pl.ds(0, C)], oh)
    plsc.subcore_barrier()
    ones16 = jnp.full((16,), 1.0, jnp.float32)
    zeros16 = jnp.full((16,), 0.0, jnp.float32)
    i16 = lax.iota(jnp.int32, 16)

    def do_chunk(m):
        off = m * C
        pltpu.sync_copy(et_hbm.at[pl.ds(off, C)], etv)
        pltpu.sync_copy(dst_hbm.at[pl.ds(off, C)], dstv)
        for k in range(C // 16):
            ets = etv[pl.ds(k * 16, 16)]
            plsc.store_scatter(oh, [i16 + k * 16, ets], ones16)
        pltpu.sync_copy(oh, cnt_sh.at[dstv], add=True)
        for k in range(C // 16):
            ets = etv[pl.ds(k * 16, 16)]
            plsc.store_scatter(oh, [i16 + k * 16, ets], zeros16)

    def chunk(j, carry):
        do_chunk(t * 78 + j)
        return carry

    lax.fori_loop(0, 78, chunk, 0)

    @pl.when(t < NCH - 32 * 78)
    def _():
        do_chunk(32 * 78 + t)

    plsc.subcore_barrier()
    pltpu.sync_copy(cnt_sh.at[pl.ds(s * TS, TS)],
                    out_hbm.at[pl.ds(c * NPAD + s * TS, TS)])


def _counts(etf, dstf, z128):
    kfn = pl.kernel(
        _counts_body,
        out_type=_f32(2 * NPAD, HH),
        mesh=_mesh(),
        scratch_types=[
            pltpu.VMEM((C,), jnp.int32),
            pltpu.VMEM((C,), jnp.int32),
            pltpu.VMEM((C, HH), jnp.float32),
            pltpu.VMEM_SHARED((NPAD, HH), jnp.float32),
        ],
        compiler_params=pltpu.CompilerParams(needs_layout_passes=False),
    )
    return kfn(etf, dstf, z128)


def _combine_body(cnt_ref, recip_ref, m_ref):
    cs = cnt_ref[...]
    tot = (cs[0] + cs[1])[:, :16]
    recip_ref[...] = 1.0 / jnp.maximum(tot, 1.0)
    deg = jnp.minimum(jnp.sum(tot, axis=1), 10.0).astype(jnp.int32)
    it = lax.broadcasted_iota(jnp.int32, (400, 128), 1)
    m_ref[...] = (deg[:, None] == it).astype(jnp.float32)


def _combine(cnt2):
    return pl.pallas_call(
        _combine_body,
        grid=(25,),
        in_specs=[pl.BlockSpec((2, 400, 128), lambda i: (0, i, 0))],
        out_specs=[
            pl.BlockSpec((400, 16), lambda i: (i, 0)),
            pl.BlockSpec((400, 128), lambda i: (i, 0)),
        ],
        out_shape=[_f32(N, 16), _f32(N, 128)],
    )(cnt2)


def _rgcn_mm_body(h_ref, w_ref, o_ref):
    hr = jnp.maximum(h_ref[...], 0.0)
    o_ref[...] = jnp.dot(hr, w_ref[...][0],
                         preferred_element_type=jnp.float32)[None, None]


def _rgcn_mm(h, w17):
    return pl.pallas_call(
        _rgcn_mm_body,
        grid=(25, 17, 2),
        in_specs=[
            pl.BlockSpec((400, 256), lambda i, j, c: (i, 0)),
            pl.BlockSpec((1, 256, 128), lambda i, j, c: (j, 0, c)),
        ],
        out_specs=pl.BlockSpec((1, 1, 400, 128), lambda i, j, c: (c, j, i, 0)),
        out_shape=_f32(2, 17, N, 128),
    )(h, w17)


def _edge_body(with_scale, *refs):
    if with_scale:
        (tab_hbm, gix_hbm, dst_hbm, seg_hbm, recip_hbm, z128_hbm,
         out_hbm,
         gix0, gix1, dstv0, dstv1, segv0, segv1, wr0, wr1,
         rows0, rows1, semA0, semA1, semB0, semB1, acc) = refs
        gix = (gix0, gix1); dstv = (dstv0, dstv1)
        segv = (segv0, segv1); wr = (wr0, wr1); rows = (rows0, rows1)
    else:
        (tab_hbm, gix_hbm, dst_hbm, z128_hbm,
         out_hbm,
         gix0, gix1, dstv0, dstv1,
         rows0, rows1, semA0, semA1, semB0, semB1, acc) = refs
        gix = (gix0, gix1); dstv = (dstv0, dstv1); rows = (rows0, rows1)
    semA = (semA0, semA1)
    semB = (semB0, semB1)

    c = lax.axis_index("c")
    s = lax.axis_index("s")
    pltpu.sync_copy(z128_hbm, acc.at[pl.ds(s * TS, TS)])
    plsc.subcore_barrier()

    def a_copies(b, k):
        off = (s * NCHT + k) * C
        yield gix_hbm.at[pl.ds(c * E + off, C)], gix[b]
        yield dst_hbm.at[pl.ds(off, C)], dstv[b]
        if with_scale:
            yield seg_hbm.at[pl.ds(off, C)], segv[b]

    def issue_a(b, k):
        for sr, dr in a_copies(b, k):
            pltpu.async_copy(sr, dr, semA[b])

    def wait_a(b):
        for sr, dr in a_copies(b, 0):
            pltpu.make_async_copy(sr, dr, semA[b]).wait()

    def b_copies(b):
        yield tab_hbm.at[gix[b]], rows[b]
        if with_scale:
            yield recip_hbm.at[segv[b]], wr[b]

    def issue_b(b):
        for sr, dr in b_copies(b):
            pltpu.async_copy(sr, dr, semB[b])

    def wait_b(b):
        for sr, dr in b_copies(b):
            pltpu.make_async_copy(sr, dr, semB[b]).wait()

    i16 = lax.iota(jnp.int32, 16)

    def scale_rows(rows_ref, wr_ref):
        def scale(e, carry):
            ef = jnp.full((16,), e, jnp.int32)
            wv = plsc.load_gather(wr_ref, [ef])
            for k in range(HH // 16):
                ci = i16 + k * 16
                v = plsc.load_gather(rows_ref, [ef, ci])
                plsc.store_scatter(rows_ref, [ef, ci], v * wv)
            return carry
        lax.fori_loop(0, C, scale, 0)

    def process(b):
        if with_scale:
            scale_rows(rows[b], wr[b])
        pltpu.sync_copy(rows[b], acc.at[dstv[b]], add=True)

    issue_a(0, 0)
    wait_a(0)
    issue_b(0)
    issue_a(1, 1)

    def pair(i, carry):
        for b in (0, 1):
            k = 2 * i + b
            ob = 1 - b
            wait_a(ob)
            issue_b(ob)
            wait_b(b)
            process(b)
            issue_a(b, k + 2)
        return carry

    lax.fori_loop(0, (NCHT - 2) // 2, pair, 0)
    wait_a(1)
    issue_b(1)
    wait_b(0)
    process(0)
    wait_b(1)
    process(1)

    @pl.when(s < NCH - 16 * NCHT)
    def _():
        off = (16 * NCHT + s) * C
        pltpu.sync_copy(gix_hbm.at[pl.ds(c * E + off, C)], gix0)
        pltpu.sync_copy(dst_hbm.at[pl.ds(off, C)], dstv0)
        if with_scale:
            pltpu.sync_copy(seg_hbm.at[pl.ds(off, C)], segv0)
            pltpu.sync_copy(recip_hbm.at[segv0], wr0)
        pltpu.sync_copy(tab_hbm.at[gix0], rows0)
        if with_scale:
            scale_rows(rows0, wr0)
        pltpu.sync_copy(rows0, acc.at[dstv0], add=True)

    plsc.subcore_barrier()
    pltpu.sync_copy(acc.at[pl.ds(s * TS, TS)],
                    out_hbm.at[pl.ds(c * NPAD + s * TS, TS)])


def _edge_pass_scaled(tab, gixf, dstf, segf, recipf, z128):
    kfn = pl.kernel(
        functools.partial(_edge_body, True),
        out_type=_f32(2 * NPAD, HH),
        mesh=_mesh(),
        scratch_types=(
            [pltpu.VMEM((C,), jnp.int32)] * 2
            + [pltpu.VMEM((C,), jnp.int32)] * 2
            + [pltpu.VMEM((C,), jnp.int32)] * 2
            + [pltpu.VMEM((C,), jnp.float32)] * 2
            + [pltpu.VMEM((C, HH), jnp.float32)] * 2
            + [pltpu.SemaphoreType.DMA] * 4
            + [pltpu.VMEM_SHARED((NPAD, HH), jnp.float32)]
        ),
        compiler_params=pltpu.CompilerParams(needs_layout_passes=False),
    )
    return kfn(tab, gixf, dstf, segf, recipf, z128)


def _edge_pass_plain(tab, gixf, dstf, z128):
    kfn = pl.kernel(
        functools.partial(_edge_body, False),
        out_type=_f32(2 * NPAD, HH),
        mesh=_mesh(),
        scratch_types=(
            [pltpu.VMEM((C,), jnp.int32)] * 2
            + [pltpu.VMEM((C,), jnp.int32)] * 2
            + [pltpu.VMEM((C, HH), jnp.float32)] * 2
            + [pltpu.SemaphoreType.DMA] * 4
            + [pltpu.VMEM_SHARED((NPAD, HH), jnp.float32)]
        ),
        compiler_params=pltpu.CompilerParams(needs_layout_passes=False),
    )
    return kfn(tab, gixf, dstf, z128)


def _post_rgcn_body(acc_ref, rt_ref, b_ref, o_ref):
    o_ref[...] = jnp.maximum(acc_ref[...] + rt_ref[...][:, 0] + b_ref[...], 0.0)


def _post_rgcn(acc2, o_all, bh):
    return pl.pallas_call(
        _post_rgcn_body,
        grid=(2, 25),
        in_specs=[
            pl.BlockSpec((1, 400, 128), lambda c, i: (c, i, 0)),
            pl.BlockSpec((1, 1, 400, 128), lambda c, i: (c, 16, i, 0)),
            pl.BlockSpec((1, 1, 128), lambda c, i: (c, 0, 0)),
        ],
        out_specs=pl.BlockSpec((1, 400, 128), lambda c, i: (c, i, 0)),
        out_shape=_f32(2, N, 128),
    )(acc2, o_all, bh)


def _mf_mm_body(hs_ref, br_ref, m_ref, w_ref, bp_ref, o_ref):
    d = pl.program_id(1)
    hs = hs_ref[...]
    br = br_ref[...]
    z = jnp.concatenate([hs[0], hs[1], br[0], br[1]], axis=1)
    mv = m_ref[...]

    @pl.when(d == 0)
    def _():
        o_ref[...] = jnp.dot(mv, bp_ref[...],
                             preferred_element_type=jnp.float32)

    it = lax.broadcasted_iota(jnp.int32, (400, 128), 1)
    md = jnp.sum(jnp.where(it == d, mv, 0.0), axis=1, keepdims=True)
    o_ref[...] += md * jnp.dot(z, w_ref[...][0],
                               preferred_element_type=jnp.float32)


def _mf_mm(hsum2, br2, m, wcat, bp):
    return pl.pallas_call(
        _mf_mm_body,
        grid=(25, ND),
        in_specs=[
            pl.BlockSpec((2, 400, 128), lambda i, d: (0, i, 0)),
            pl.BlockSpec((2, 400, 128), lambda i, d: (0, i, 0)),
            pl.BlockSpec((400, 128), lambda i, d: (i, 0)),
            pl.BlockSpec((1, 512, 256), lambda i, d: (d, 0, 0)),
            pl.BlockSpec((128, 256), lambda i, d: (0, 0)),
        ],
        out_specs=pl.BlockSpec((400, 256), lambda i, d: (i, 0)),
        out_shape=_f32(N, 256),
    )(hsum2, br2, m, wcat, bp)


def _pool_body(h_ref, bi_ref, w1_ref, b1_ref, w2_ref, b2_ref, o_ref, acc):
    i = pl.program_id(0)

    @pl.when(i == 0)
    def _():
        acc[...] = jnp.zeros_like(acc)

    bi = bi_ref[...][0]
    it = lax.broadcasted_iota(jnp.int32, (G, 400), 0)
    mask = (it == bi).astype(jnp.float32)
    acc[...] += jnp.dot(mask, h_ref[...], preferred_element_type=jnp.float32)

    @pl.when(i == 24)
    def _():
        y = acc[...]
        t = jnp.maximum(jnp.dot(y, w1_ref[...],
                                preferred_element_type=jnp.float32)
                        + b1_ref[...], 0.0)
        o_ref[...] = jnp.dot(t, w2_ref[...],
                             preferred_element_type=jnp.float32) + b2_ref[...]


def _pool_head(h, bidx3, w1, b1, w2, b2):
    return pl.pallas_call(
        _pool_body,
        grid=(25,),
        in_specs=[
            pl.BlockSpec((400, 256), lambda i: (i, 0)),
            pl.BlockSpec((1, 1, 400), lambda i: (i, 0, 0)),
            pl.BlockSpec((256, 256), lambda i: (0, 0)),
            pl.BlockSpec((1, 256), lambda i: (0, 0)),
            pl.BlockSpec((256, 128), lambda i: (0, 0)),
            pl.BlockSpec((1, 128), lambda i: (0, 0)),
        ],
        out_specs=pl.BlockSpec((G, 128), lambda i: (0, 0)),
        out_shape=_f32(G, 128),
        scratch_shapes=[pltpu.VMEM((G, 256), jnp.float32)],
    )(h, bidx3, w1, b1, w2, b2)


def kernel(x, edge_index, edge_attr, batch_idx, W_emb, b_emb, rgcn_w,
           rgcn_root, rgcn_b, mf_w, mf_b, mf_root, W1, b1, W2, b2):
    src = edge_index[0]
    dst = edge_index[1]
    srcm = src.reshape(20, 1, 16000)
    edge_attr_t = edge_attr.T
    dstf = dst

    dstm = dst.reshape(20, 1, 16000)
    etm, segm, gix2, src2 = _prep(edge_attr_t, srcm, dstm)
    etf = etm.reshape(E)
    segf = segm.reshape(E)
    gixf = gix2.reshape(2 * E)
    src2f = src2.reshape(2 * E)

    z128 = jnp.zeros((TS, HH), jnp.float32)

    cnt2 = _counts(etf, dstf, z128).reshape(2, NPAD, HH)
    recip, m_onehot = _combine(cnt2)

    w17 = jnp.concatenate([rgcn_w, rgcn_root[:, None]], axis=1)
    wcat = jnp.concatenate([mf_w, mf_root], axis=2)
    bp = jnp.pad(mf_b, ((0, 0), (0, 128 - ND), (0, 0)))
    bh = rgcn_b.reshape(2, 2, 1, 128)

    h = _emb(x, W_emb, b_emb.reshape(1, 256))
    for blk in range(2):
        o_all = _rgcn_mm(h, w17[blk])
        xt = o_all.reshape(2 * XR, HH)
        acc2 = _edge_pass_scaled(xt, gixf, dstf, segf, recip.reshape(N * R),
                                 z128).reshape(2, NPAD, 128)
        br2 = _post_rgcn(acc2, o_all, bh[blk])
        hsum2 = _edge_pass_plain(br2.reshape(2 * N, HH), src2f, dstf,
                                 z128).reshape(2, NPAD, 128)
        h = _mf_mm(hsum2, br2.reshape(2, N, 128), m_onehot, wcat[blk], bp[blk])

    out = _pool_head(h, batch_idx.reshape(25, 1, 400), W1, b1.reshape(1, 256),
                     W2, b2.reshape(1, 128))
    return out


# speedup vs baseline: 3.0251x; 1.4144x over previous
import functools

import jax
import jax.numpy as jnp
from jax import lax
from jax.experimental import pallas as pl
from jax.experimental.pallas import tpu as pltpu
from jax.experimental.pallas import tpu_sc as plsc

N = 10000
E = 320000
R = 16
H = 256
HH = 128
G = 64
ND = 11
XR = 17 * N
NPAD = 10240
TS = NPAD // 16

C = 128
NCH = E // C
NCHT = 156

@functools.cache
def _mesh():
    return plsc.VectorSubcoreMesh(core_axis_name="c", subcore_axis_name="s")


def _f32(*shape):
    return jax.ShapeDtypeStruct(shape, jnp.float32)


def _i32(*shape):
    return jax.ShapeDtypeStruct(shape, jnp.int32)


def _prep_body(ea_ref, src_ref, dst_ref, et_ref, seg_ref, gix2_ref, src2_ref):
    ea = ea_ref[...]
    m = jnp.max(ea, axis=0, keepdims=True)
    it = lax.broadcasted_iota(jnp.int32, ea.shape, 0)
    et = jnp.min(jnp.where(ea == m, it, R), axis=0)
    src = src_ref[...][0, 0]
    dst = dst_ref[...][0, 0]
    et_ref[...] = et[None, None]
    seg_ref[...] = (dst * R + et)[None, None]
    gix = et * N + src
    gix2_ref[...] = jnp.stack([gix, gix + XR], axis=0)[:, None, None]
    src2_ref[...] = jnp.stack([src, src + N], axis=0)[:, None, None]


def _prep(edge_attr_t, srcm, dstm):
    cb = 16000
    return pl.pallas_call(
        _prep_body,
        grid=(20,),
        in_specs=[
            pl.BlockSpec((16, cb), lambda i: (0, i)),
            pl.BlockSpec((1, 1, cb), lambda i: (i, 0, 0)),
            pl.BlockSpec((1, 1, cb), lambda i: (i, 0, 0)),
        ],
        out_specs=[
            pl.BlockSpec((1, 1, cb), lambda i: (i, 0, 0)),
            pl.BlockSpec((1, 1, cb), lambda i: (i, 0, 0)),
            pl.BlockSpec((2, 1, 1, cb), lambda i: (0, i, 0, 0)),
            pl.BlockSpec((2, 1, 1, cb), lambda i: (0, i, 0, 0)),
        ],
        out_shape=[_i32(20, 1, cb), _i32(20, 1, cb), _i32(2, 20, 1, cb),
                   _i32(2, 20, 1, cb)],
    )(edge_attr_t, srcm, dstm)


def _emb_body(x_ref, w_ref, b_ref, o_ref):
    o_ref[...] = jnp.dot(x_ref[...], w_ref[...],
                         preferred_element_type=jnp.float32) + b_ref[...]


def _emb(x, w, b):
    return pl.pallas_call(
        _emb_body,
        grid=(25,),
        in_specs=[
            pl.BlockSpec((400, 128), lambda i: (i, 0)),
            pl.BlockSpec((128, 256), lambda i: (0, 0)),
            pl.BlockSpec((1, 256), lambda i: (0, 0)),
        ],
        out_specs=pl.BlockSpec((400, 256), lambda i: (i, 0)),
        out_shape=_f32(N, 256),
    )(x, w, b)


def _counts_body(et_hbm, dst_hbm, z128_hbm, out_hbm, etv, dstv, oh, cnt_sh):
    c = lax.axis_index("c")
    s = lax.axis_index("s")
    t = c * 16 + s
    pltpu.sync_copy(z128_hbm, cnt_sh.at[pl.ds(s * TS, TS)])
    pltpu.sync_copy(z128_hbm.at[pl.ds(0, C)], oh)
    plsc.subcore_barrier()
    ones16 = jnp.full((16,), 1.0, jnp.float32)
    zeros16 = jnp.full((16,), 0.0, jnp.float32)
    i16 = lax.iota(jnp.int32, 16)

    def do_chunk(m):
        off = m * C
        pltpu.sync_copy(et_hbm.at[pl.ds(off, C)], etv)
        pltpu.sync_copy(dst_hbm.at[pl.ds(off, C)], dstv)
        for k in range(C // 16):
            ets = etv[pl.ds(k * 16, 16)]
            plsc.store_scatter(oh, [i16 + k * 16, ets], ones16)
        pltpu.sync_copy(oh, cnt_sh.at[dstv], add=True)
        for k in range(C // 16):
            ets = etv[pl.ds(k * 16, 16)]
            plsc.store_scatter(oh, [i16 + k * 16, ets], zeros16)

    def chunk(j, carry):
        do_chunk(t * 78 + j)
        return carry

    lax.fori_loop(0, 78, chunk, 0)

    @pl.when(t < NCH - 32 * 78)
    def _():
        do_chunk(32 * 78 + t)

    plsc.subcore_barrier()
    pltpu.sync_copy(cnt_sh.at[pl.ds(s * TS, TS)],
                    out_hbm.at[pl.ds(c * NPAD + s * TS, TS)])


def _counts(etf, dstf, z128):
    kfn = pl.kernel(
        _counts_body,
        out_type=_f32(2 * NPAD, HH),
        mesh=_mesh(),
        scratch_types=[
            pltpu.VMEM((C,), jnp.int32),
            pltpu.VMEM((C,), jnp.int32),
            pltpu.VMEM((C, HH), jnp.float32),
            pltpu.VMEM_SHARED((NPAD, HH), jnp.float32),
        ],
        compiler_params=pltpu.CompilerParams(needs_layout_passes=False),
    )
    return kfn(etf, dstf, z128)


def _combine_body(cnt_ref, recip_ref, m_ref):
    cs = cnt_ref[...]
    tot = (cs[0] + cs[1])[:, :16]
    recip_ref[...] = 1.0 / jnp.maximum(tot, 1.0)
    deg = jnp.minimum(jnp.sum(tot, axis=1), 10.0).astype(jnp.int32)
    it = lax.broadcasted_iota(jnp.int32, (400, 128), 1)
    m_ref[...] = (deg[:, None] == it).astype(jnp.float32)


def _combine(cnt2):
    return pl.pallas_call(
        _combine_body,
        grid=(25,),
        in_specs=[pl.BlockSpec((2, 400, 128), lambda i: (0, i, 0))],
        out_specs=[
            pl.BlockSpec((400, 16), lambda i: (i, 0)),
            pl.BlockSpec((400, 128), lambda i: (i, 0)),
        ],
        out_shape=[_f32(N, 16), _f32(N, 128)],
    )(cnt2)


def _rgcn_mm_body(h_ref, w_ref, o_ref):
    hr = jnp.maximum(h_ref[...], 0.0).astype(jnp.bfloat16)
    o_ref[...] = jnp.dot(hr, w_ref[...][0].astype(jnp.bfloat16),
                         preferred_element_type=jnp.float32)[None, None]


def _rgcn_mm(h, w17):
    return pl.pallas_call(
        _rgcn_mm_body,
        grid=(25, 17, 2),
        in_specs=[
            pl.BlockSpec((400, 256), lambda i, j, c: (i, 0)),
            pl.BlockSpec((1, 256, 128), lambda i, j, c: (j, 0, c)),
        ],
        out_specs=pl.BlockSpec((1, 1, 400, 128), lambda i, j, c: (c, j, i, 0)),
        out_shape=_f32(2, 17, N, 128),
    )(h, w17)


def _edge_body(with_scale, *refs):
    if with_scale:
        (tab_hbm, gix_hbm, dst_hbm, seg_hbm, recip_hbm, z128_hbm,
         out_hbm,
         gix0, gix1, dstv0, dstv1, segv0, segv1, wr0, wr1,
         rows0, rows1, semA0, semA1, semB0, semB1, acc) = refs
        gix = (gix0, gix1); dstv = (dstv0, dstv1)
        segv = (segv0, segv1); wr = (wr0, wr1); rows = (rows0, rows1)
    else:
        (tab_hbm, gix_hbm, dst_hbm, z128_hbm,
         out_hbm,
         gix0, gix1, dstv0, dstv1,
         rows0, rows1, semA0, semA1, semB0, semB1, acc) = refs
        gix = (gix0, gix1); dstv = (dstv0, dstv1); rows = (rows0, rows1)
    semA = (semA0, semA1)
    semB = (semB0, semB1)

    c = lax.axis_index("c")
    s = lax.axis_index("s")
    pltpu.sync_copy(z128_hbm, acc.at[pl.ds(s * TS, TS)])
    plsc.subcore_barrier()

    def a_copies(b, k):
        off = (s * NCHT + k) * C
        yield gix_hbm.at[pl.ds(c * E + off, C)], gix[b]
        yield dst_hbm.at[pl.ds(off, C)], dstv[b]
        if with_scale:
            yield seg_hbm.at[pl.ds(off, C)], segv[b]

    def issue_a(b, k):
        for sr, dr in a_copies(b, k):
            pltpu.async_copy(sr, dr, semA[b])

    def wait_a(b):
        for sr, dr in a_copies(b, 0):
            pltpu.make_async_copy(sr, dr, semA[b]).wait()

    def b_copies(b):
        yield tab_hbm.at[gix[b]], rows[b]
        if with_scale:
            yield recip_hbm.at[segv[b]], wr[b]

    def issue_b(b):
        for sr, dr in b_copies(b):
            pltpu.async_copy(sr, dr, semB[b])

    def wait_b(b):
        for sr, dr in b_copies(b):
            pltpu.make_async_copy(sr, dr, semB[b]).wait()

    i16 = lax.iota(jnp.int32, 16)

    def scale_rows(rows_ref, wr_ref):
        def scale(e, carry):
            ef = jnp.full((16,), e, jnp.int32)
            wv = plsc.load_gather(wr_ref, [ef])
            for k in range(HH // 16):
                sl = pl.ds(k * 16, 16)
                rows_ref[e, sl] = rows_ref[e, sl] * wv
            return carry
        lax.fori_loop(0, C, scale, 0)

    def process(b):
        if with_scale:
            scale_rows(rows[b], wr[b])
        pltpu.sync_copy(rows[b], acc.at[dstv[b]], add=True)

    issue_a(0, 0)
    wait_a(0)
    issue_b(0)
    issue_a(1, 1)

    def pair(i, carry):
        for b in (0, 1):
            k = 2 * i + b
            ob = 1 - b
            wait_a(ob)
            issue_b(ob)
            wait_b(b)
            process(b)
            issue_a(b, k + 2)
        return carry

    lax.fori_loop(0, (NCHT - 2) // 2, pair, 0)
    wait_a(1)
    issue_b(1)
    wait_b(0)
    process(0)
    wait_b(1)
    process(1)

    @pl.when(s < NCH - 16 * NCHT)
    def _():
        off = (16 * NCHT + s) * C
        pltpu.sync_copy(gix_hbm.at[pl.ds(c * E + off, C)], gix0)
        pltpu.sync_copy(dst_hbm.at[pl.ds(off, C)], dstv0)
        if with_scale:
            pltpu.sync_copy(seg_hbm.at[pl.ds(off, C)], segv0)
            pltpu.sync_copy(recip_hbm.at[segv0], wr0)
        pltpu.sync_copy(tab_hbm.at[gix0], rows0)
        if with_scale:
            scale_rows(rows0, wr0)
        pltpu.sync_copy(rows0, acc.at[dstv0], add=True)

    plsc.subcore_barrier()
    pltpu.sync_copy(acc.at[pl.ds(s * TS, TS)],
                    out_hbm.at[pl.ds(c * NPAD + s * TS, TS)])


def _edge_pass_scaled(tab, gixf, dstf, segf, recipf, z128):
    kfn = pl.kernel(
        functools.partial(_edge_body, True),
        out_type=_f32(2 * NPAD, HH),
        mesh=_mesh(),
        scratch_types=(
            [pltpu.VMEM((C,), jnp.int32)] * 2
            + [pltpu.VMEM((C,), jnp.int32)] * 2
            + [pltpu.VMEM((C,), jnp.int32)] * 2
            + [pltpu.VMEM((C,), jnp.float32)] * 2
            + [pltpu.VMEM((C, HH), jnp.float32)] * 2
            + [pltpu.SemaphoreType.DMA] * 4
            + [pltpu.VMEM_SHARED((NPAD, HH), jnp.float32)]
        ),
        compiler_params=pltpu.CompilerParams(needs_layout_passes=False),
    )
    return kfn(tab, gixf, dstf, segf, recipf, z128)


def _edge_pass_plain(tab, gixf, dstf, z128):
    kfn = pl.kernel(
        functools.partial(_edge_body, False),
        out_type=_f32(2 * NPAD, HH),
        mesh=_mesh(),
        scratch_types=(
            [pltpu.VMEM((C,), jnp.int32)] * 2
            + [pltpu.VMEM((C,), jnp.int32)] * 2
            + [pltpu.VMEM((C, HH), jnp.float32)] * 2
            + [pltpu.SemaphoreType.DMA] * 4
            + [pltpu.VMEM_SHARED((NPAD, HH), jnp.float32)]
        ),
        compiler_params=pltpu.CompilerParams(needs_layout_passes=False),
    )
    return kfn(tab, gixf, dstf, z128)


def _post_rgcn_body(acc_ref, rt_ref, b_ref, o_ref):
    o_ref[...] = jnp.maximum(acc_ref[...] + rt_ref[...][:, 0] + b_ref[...], 0.0)


def _post_rgcn(acc2, o_all, bh):
    return pl.pallas_call(
        _post_rgcn_body,
        grid=(2, 25),
        in_specs=[
            pl.BlockSpec((1, 400, 128), lambda c, i: (c, i, 0)),
            pl.BlockSpec((1, 1, 400, 128), lambda c, i: (c, 16, i, 0)),
            pl.BlockSpec((1, 1, 128), lambda c, i: (c, 0, 0)),
        ],
        out_specs=pl.BlockSpec((1, 400, 128), lambda c, i: (c, i, 0)),
        out_shape=_f32(2, N, 128),
    )(acc2, o_all, bh)


def _mf_mm_body(hs_ref, br_ref, m_ref, w_ref, bp_ref, o_ref):
    d = pl.program_id(1)
    hs = hs_ref[...]
    br = br_ref[...]
    z = jnp.concatenate([hs[0], hs[1], br[0], br[1]], axis=1)
    mv = m_ref[...]

    @pl.when(d == 0)
    def _():
        o_ref[...] = jnp.dot(mv, bp_ref[...],
                             preferred_element_type=jnp.float32)

    it = lax.broadcasted_iota(jnp.int32, (400, 128), 1)
    md = jnp.sum(jnp.where(it == d, mv, 0.0), axis=1, keepdims=True)
    o_ref[...] += md * jnp.dot(z.astype(jnp.bfloat16),
                               w_ref[...][0].astype(jnp.bfloat16),
                               preferred_element_type=jnp.float32)


def _mf_mm(hsum2, br2, m, wcat, bp):
    return pl.pallas_call(
        _mf_mm_body,
        grid=(25, ND),
        in_specs=[
            pl.BlockSpec((2, 400, 128), lambda i, d: (0, i, 0)),
            pl.BlockSpec((2, 400, 128), lambda i, d: (0, i, 0)),
            pl.BlockSpec((400, 128), lambda i, d: (i, 0)),
            pl.BlockSpec((1, 512, 256), lambda i, d: (d, 0, 0)),
            pl.BlockSpec((128, 256), lambda i, d: (0, 0)),
        ],
        out_specs=pl.BlockSpec((400, 256), lambda i, d: (i, 0)),
        out_shape=_f32(N, 256),
    )(hsum2, br2, m, wcat, bp)


def _pool_body(h_ref, bi_ref, w1_ref, b1_ref, w2_ref, b2_ref, o_ref, acc):
    i = pl.program_id(0)

    @pl.when(i == 0)
    def _():
        acc[...] = jnp.zeros_like(acc)

    bi = bi_ref[...][0]
    it = lax.broadcasted_iota(jnp.int32, (G, 400), 0)
    mask = (it == bi).astype(jnp.float32)
    acc[...] += jnp.dot(mask, h_ref[...], preferred_element_type=jnp.float32)

    @pl.when(i == 24)
    def _():
        y = acc[...]
        t = jnp.maximum(jnp.dot(y, w1_ref[...],
                                preferred_element_type=jnp.float32)
                        + b1_ref[...], 0.0)
        o_ref[...] = jnp.dot(t, w2_ref[...],
                             preferred_element_type=jnp.float32) + b2_ref[...]


def _pool_head(h, bidx3, w1, b1, w2, b2):
    return pl.pallas_call(
        _pool_body,
        grid=(25,),
        in_specs=[
            pl.BlockSpec((400, 256), lambda i: (i, 0)),
            pl.BlockSpec((1, 1, 400), lambda i: (i, 0, 0)),
            pl.BlockSpec((256, 256), lambda i: (0, 0)),
            pl.BlockSpec((1, 256), lambda i: (0, 0)),
            pl.BlockSpec((256, 128), lambda i: (0, 0)),
            pl.BlockSpec((1, 128), lambda i: (0, 0)),
        ],
        out_specs=pl.BlockSpec((G, 128), lambda i: (0, 0)),
        out_shape=_f32(G, 128),
        scratch_shapes=[pltpu.VMEM((G, 256), jnp.float32)],
    )(h, bidx3, w1, b1, w2, b2)


def kernel(x, edge_index, edge_attr, batch_idx, W_emb, b_emb, rgcn_w,
           rgcn_root, rgcn_b, mf_w, mf_b, mf_root, W1, b1, W2, b2):
    src = edge_index[0]
    dst = edge_index[1]
    srcm = src.reshape(20, 1, 16000)
    edge_attr_t = edge_attr.T
    dstf = dst

    dstm = dst.reshape(20, 1, 16000)
    etm, segm, gix2, src2 = _prep(edge_attr_t, srcm, dstm)
    etf = etm.reshape(E)
    segf = segm.reshape(E)
    gixf = gix2.reshape(2 * E)
    src2f = src2.reshape(2 * E)

    z128 = jnp.zeros((TS, HH), jnp.float32)

    cnt2 = _counts(etf, dstf, z128).reshape(2, NPAD, HH)
    recip, m_onehot = _combine(cnt2)

    w17 = jnp.concatenate([rgcn_w, rgcn_root[:, None]], axis=1)
    wcat = jnp.concatenate([mf_w, mf_root], axis=2)
    bp = jnp.pad(mf_b, ((0, 0), (0, 128 - ND), (0, 0)))
    bh = rgcn_b.reshape(2, 2, 1, 128)

    h = _emb(x, W_emb, b_emb.reshape(1, 256))
    for blk in range(2):
        o_all = _rgcn_mm(h, w17[blk])
        xt = o_all.reshape(2 * XR, HH)
        acc2 = _edge_pass_scaled(xt, gixf, dstf, segf, recip.reshape(N * R),
                                 z128).reshape(2, NPAD, 128)
        br2 = _post_rgcn(acc2, o_all, bh[blk])
        hsum2 = _edge_pass_plain(br2.reshape(2 * N, HH), src2f, dstf,
                                 z128).reshape(2, NPAD, 128)
        h = _mf_mm(hsum2, br2.reshape(2, N, 128), m_onehot, wcat[blk], bp[blk])

    out = _pool_head(h, batch_idx.reshape(25, 1, 400), W1, b1.reshape(1, 256),
                     W2, b2.reshape(1, 128))
    return out


# speedup vs baseline: 3.2633x; 1.0787x over previous
import functools

import jax
import jax.numpy as jnp
from jax import lax
from jax.experimental import pallas as pl
from jax.experimental.pallas import tpu as pltpu
from jax.experimental.pallas import tpu_sc as plsc

N = 10000
E = 320000
R = 16
H = 256
HH = 128
G = 64
ND = 11
XR = 17 * N
NPAD = 10240
TS = NPAD // 16

C = 128
NCH = E // C
NCHT = 156

@functools.cache
def _mesh():
    return plsc.VectorSubcoreMesh(core_axis_name="c", subcore_axis_name="s")


def _f32(*shape):
    return jax.ShapeDtypeStruct(shape, jnp.float32)


def _i32(*shape):
    return jax.ShapeDtypeStruct(shape, jnp.int32)


def _prep_body(ea_ref, src_ref, dst_ref, et_ref, seg_ref, gix2_ref, src2_ref):
    ea = ea_ref[...]
    m = jnp.max(ea, axis=0, keepdims=True)
    it = lax.broadcasted_iota(jnp.int32, ea.shape, 0)
    et = jnp.min(jnp.where(ea == m, it, R), axis=0)
    src = src_ref[...][0, 0]
    dst = dst_ref[...][0, 0]
    et_ref[...] = et[None, None]
    seg_ref[...] = (dst * R + et)[None, None]
    gix = et * N + src
    gix2_ref[...] = jnp.stack([gix, gix + XR], axis=0)[:, None, None]
    src2_ref[...] = jnp.stack([src, src + N], axis=0)[:, None, None]


def _prep(edge_attr_t, srcm, dstm):
    cb = 16000
    return pl.pallas_call(
        _prep_body,
        grid=(20,),
        in_specs=[
            pl.BlockSpec((16, cb), lambda i: (0, i)),
            pl.BlockSpec((1, 1, cb), lambda i: (i, 0, 0)),
            pl.BlockSpec((1, 1, cb), lambda i: (i, 0, 0)),
        ],
        out_specs=[
            pl.BlockSpec((1, 1, cb), lambda i: (i, 0, 0)),
            pl.BlockSpec((1, 1, cb), lambda i: (i, 0, 0)),
            pl.BlockSpec((2, 1, 1, cb), lambda i: (0, i, 0, 0)),
            pl.BlockSpec((2, 1, 1, cb), lambda i: (0, i, 0, 0)),
        ],
        out_shape=[_i32(20, 1, cb), _i32(20, 1, cb), _i32(2, 20, 1, cb),
                   _i32(2, 20, 1, cb)],
    )(edge_attr_t, srcm, dstm)


def _emb_body(x_ref, w_ref, b_ref, o_ref):
    o_ref[...] = jnp.dot(x_ref[...], w_ref[...],
                         preferred_element_type=jnp.float32) + b_ref[...]


def _emb(x, w, b):
    return pl.pallas_call(
        _emb_body,
        grid=(25,),
        in_specs=[
            pl.BlockSpec((400, 128), lambda i: (i, 0)),
            pl.BlockSpec((128, 256), lambda i: (0, 0)),
            pl.BlockSpec((1, 256), lambda i: (0, 0)),
        ],
        out_specs=pl.BlockSpec((400, 256), lambda i: (i, 0)),
        out_shape=_f32(N, 256),
    )(x, w, b)


def _counts_body(et_hbm, dst_hbm, z128_hbm, out_hbm, etv, dstv, oh, cnt_sh):
    c = lax.axis_index("c")
    s = lax.axis_index("s")
    t = c * 16 + s
    pltpu.sync_copy(z128_hbm, cnt_sh.at[pl.ds(s * TS, TS)])
    pltpu.sync_copy(z128_hbm.at[pl.ds(0, C)], oh)
    plsc.subcore_barrier()
    ones16 = jnp.full((16,), 1.0, jnp.float32)
    zeros16 = jnp.full((16,), 0.0, jnp.float32)
    i16 = lax.iota(jnp.int32, 16)

    def do_chunk(m):
        off = m * C
        pltpu.sync_copy(et_hbm.at[pl.ds(off, C)], etv)
        pltpu.sync_copy(dst_hbm.at[pl.ds(off, C)], dstv)
        for k in range(C // 16):
            ets = etv[pl.ds(k * 16, 16)]
            plsc.store_scatter(oh, [i16 + k * 16, ets], ones16)
        pltpu.sync_copy(oh, cnt_sh.at[dstv], add=True)
        for k in range(C // 16):
            ets = etv[pl.ds(k * 16, 16)]
            plsc.store_scatter(oh, [i16 + k * 16, ets], zeros16)

    def chunk(j, carry):
        do_chunk(t * 78 + j)
        return carry

    lax.fori_loop(0, 78, chunk, 0)

    @pl.when(t < NCH - 32 * 78)
    def _():
        do_chunk(32 * 78 + t)

    plsc.subcore_barrier()
    pltpu.sync_copy(cnt_sh.at[pl.ds(s * TS, TS)],
                    out_hbm.at[pl.ds(c * NPAD + s * TS, TS)])


def _counts(etf, dstf, z128):
    kfn = pl.kernel(
        _counts_body,
        out_type=_f32(2 * NPAD, HH),
        mesh=_mesh(),
        scratch_types=[
            pltpu.VMEM((C,), jnp.int32),
            pltpu.VMEM((C,), jnp.int32),
            pltpu.VMEM((C, HH), jnp.float32),
            pltpu.VMEM_SHARED((NPAD, HH), jnp.float32),
        ],
        compiler_params=pltpu.CompilerParams(needs_layout_passes=False),
    )
    return kfn(etf, dstf, z128)


def _combine_body(cnt_ref, recip_ref, m_ref):
    cs = cnt_ref[...]
    tot = (cs[0] + cs[1])[:, :16]
    recip_ref[...] = 1.0 / jnp.maximum(tot, 1.0)
    deg = jnp.minimum(jnp.sum(tot, axis=1), 10.0).astype(jnp.int32)
    it = lax.broadcasted_iota(jnp.int32, (400, 128), 1)
    m_ref[...] = (deg[:, None] == it).astype(jnp.float32)


def _combine(cnt2):
    return pl.pallas_call(
        _combine_body,
        grid=(25,),
        in_specs=[pl.BlockSpec((2, 400, 128), lambda i: (0, i, 0))],
        out_specs=[
            pl.BlockSpec((400, 16), lambda i: (i, 0)),
            pl.BlockSpec((400, 128), lambda i: (i, 0)),
        ],
        out_shape=[_f32(N, 16), _f32(N, 128)],
    )(cnt2)


def _rgcn_mm_body(h_ref, w_ref, o_ref):
    hr = jnp.maximum(h_ref[...], 0.0).astype(jnp.bfloat16)
    o_ref[...] = jnp.dot(hr, w_ref[...][0].astype(jnp.bfloat16),
                         preferred_element_type=jnp.float32)[None, None]


def _rgcn_mm(h, w17):
    return pl.pallas_call(
        _rgcn_mm_body,
        grid=(25, 17, 2),
        in_specs=[
            pl.BlockSpec((400, 256), lambda i, j, c: (i, 0)),
            pl.BlockSpec((1, 256, 128), lambda i, j, c: (j, 0, c)),
        ],
        out_specs=pl.BlockSpec((1, 1, 400, 128), lambda i, j, c: (c, j, i, 0)),
        out_shape=_f32(2, 17, N, 128),
    )(h, w17)


def _edge_body(with_scale, *refs):
    if with_scale:
        (tab_hbm, gix_hbm, dst_hbm, seg_hbm, recip_hbm, z128_hbm,
         out_hbm,
         gix0, gix1, dstv0, dstv1, segv0, segv1, wr0, wr1,
         rows0, rows1, dsc0, dsc1,
         semA0, semA1, semB0, semB1, semC0, semC1, acc) = refs
        gix = (gix0, gix1); dstv = (dstv0, dstv1)
        segv = (segv0, segv1); wr = (wr0, wr1); rows = (rows0, rows1)
    else:
        (tab_hbm, gix_hbm, dst_hbm, z128_hbm,
         out_hbm,
         gix0, gix1, dstv0, dstv1,
         rows0, rows1, dsc0, dsc1,
         semA0, semA1, semB0, semB1, semC0, semC1, acc) = refs
        gix = (gix0, gix1); dstv = (dstv0, dstv1); rows = (rows0, rows1)
    semA = (semA0, semA1)
    semB = (semB0, semB1)
    semC = (semC0, semC1)
    dsc = (dsc0, dsc1)

    c = lax.axis_index("c")
    s = lax.axis_index("s")
    pltpu.sync_copy(z128_hbm, acc.at[pl.ds(s * TS, TS)])
    plsc.subcore_barrier()

    def a_copies(b, k):
        off = (s * NCHT + k) * C
        yield gix_hbm.at[pl.ds(c * E + off, C)], gix[b]
        yield dst_hbm.at[pl.ds(off, C)], dstv[b]
        if with_scale:
            yield seg_hbm.at[pl.ds(off, C)], segv[b]

    def issue_a(b, k):
        for sr, dr in a_copies(b, k):
            pltpu.async_copy(sr, dr, semA[b])

    def wait_a(b):
        for sr, dr in a_copies(b, 0):
            pltpu.make_async_copy(sr, dr, semA[b]).wait()

    def b_copies(b):
        yield tab_hbm.at[gix[b]], rows[b]
        if with_scale:
            yield recip_hbm.at[segv[b]], wr[b]

    def issue_b(b):
        for sr, dr in b_copies(b):
            pltpu.async_copy(sr, dr, semB[b])

    def wait_b(b):
        for sr, dr in b_copies(b):
            pltpu.make_async_copy(sr, dr, semB[b]).wait()

    i16 = lax.iota(jnp.int32, 16)

    def scale_rows(rows_ref, wr_ref):
        def scale(e, carry):
            ef = jnp.full((16,), e, jnp.int32)
            wv = plsc.load_gather(wr_ref, [ef])
            for k in range(HH // 16):
                sl = pl.ds(k * 16, 16)
                rows_ref[e, sl] = rows_ref[e, sl] * wv
            return carry
        lax.fori_loop(0, C, scale, 0)

    def issue_c(b):
        for k in range(C // 16):
            sl = pl.ds(k * 16, 16)
            dsc[b][sl] = dstv[b][sl]
        pltpu.async_copy(rows[b], acc.at[dsc[b]], semC[b], add=True)

    def wait_c(b):
        pltpu.make_async_copy(rows[b], acc.at[dsc[b]], semC[b]).wait()

    def process(b):
        if with_scale:
            scale_rows(rows[b], wr[b])
        issue_c(b)

    issue_a(0, 0)
    wait_a(0)
    issue_b(0)
    issue_a(1, 1)

    def pair(i, carry):
        for b in (0, 1):
            k = 2 * i + b
            ob = 1 - b
            wait_a(ob)

            @pl.when(k > 0)
            def _():
                wait_c(ob)
            issue_b(ob)
            wait_b(b)
            process(b)
            issue_a(b, k + 2)
        return carry

    lax.fori_loop(0, (NCHT - 2) // 2, pair, 0)
    wait_a(1)
    wait_c(1)
    issue_b(1)
    wait_b(0)
    process(0)
    wait_b(1)
    process(1)
    wait_c(0)
    wait_c(1)

    @pl.when(s < NCH - 16 * NCHT)
    def _():
        off = (16 * NCHT + s) * C
        pltpu.sync_copy(gix_hbm.at[pl.ds(c * E + off, C)], gix0)
        pltpu.sync_copy(dst_hbm.at[pl.ds(off, C)], dstv0)
        if with_scale:
            pltpu.sync_copy(seg_hbm.at[pl.ds(off, C)], segv0)
            pltpu.sync_copy(recip_hbm.at[segv0], wr0)
        pltpu.sync_copy(tab_hbm.at[gix0], rows0)
        if with_scale:
            scale_rows(rows0, wr0)
        pltpu.sync_copy(rows0, acc.at[dstv0], add=True)

    plsc.subcore_barrier()
    pltpu.sync_copy(acc.at[pl.ds(s * TS, TS)],
                    out_hbm.at[pl.ds(c * NPAD + s * TS, TS)])


def _edge_pass_scaled(tab, gixf, dstf, segf, recipf, z128):
    kfn = pl.kernel(
        functools.partial(_edge_body, True),
        out_type=_f32(2 * NPAD, HH),
        mesh=_mesh(),
        scratch_types=(
            [pltpu.VMEM((C,), jnp.int32)] * 2
            + [pltpu.VMEM((C,), jnp.int32)] * 2
            + [pltpu.VMEM((C,), jnp.int32)] * 2
            + [pltpu.VMEM((C,), jnp.float32)] * 2
            + [pltpu.VMEM((C, HH), jnp.float32)] * 2
            + [pltpu.VMEM((C,), jnp.int32)] * 2
            + [pltpu.SemaphoreType.DMA] * 6
            + [pltpu.VMEM_SHARED((NPAD, HH), jnp.float32)]
        ),
        compiler_params=pltpu.CompilerParams(needs_layout_passes=False),
    )
    return kfn(tab, gixf, dstf, segf, recipf, z128)


def _edge_pass_plain(tab, gixf, dstf, z128):
    kfn = pl.kernel(
        functools.partial(_edge_body, False),
        out_type=_f32(2 * NPAD, HH),
        mesh=_mesh(),
        scratch_types=(
            [pltpu.VMEM((C,), jnp.int32)] * 2
            + [pltpu.VMEM((C,), jnp.int32)] * 2
            + [pltpu.VMEM((C, HH), jnp.float32)] * 2
            + [pltpu.VMEM((C,), jnp.int32)] * 2
            + [pltpu.SemaphoreType.DMA] * 6
            + [pltpu.VMEM_SHARED((NPAD, HH), jnp.float32)]
        ),
        compiler_params=pltpu.CompilerParams(needs_layout_passes=False),
    )
    return kfn(tab, gixf, dstf, z128)


def _post_rgcn_body(acc_ref, rt_ref, b_ref, o_ref):
    o_ref[...] = jnp.maximum(acc_ref[...] + rt_ref[...][:, 0] + b_ref[...], 0.0)


def _post_rgcn(acc2, o_all, bh):
    return pl.pallas_call(
        _post_rgcn_body,
        grid=(2, 25),
        in_specs=[
            pl.BlockSpec((1, 400, 128), lambda c, i: (c, i, 0)),
            pl.BlockSpec((1, 1, 400, 128), lambda c, i: (c, 16, i, 0)),
            pl.BlockSpec((1, 1, 128), lambda c, i: (c, 0, 0)),
        ],
        out_specs=pl.BlockSpec((1, 400, 128), lambda c, i: (c, i, 0)),
        out_shape=_f32(2, N, 128),
    )(acc2, o_all, bh)


def _mf_mm_body(hs_ref, br_ref, m_ref, w_ref, bp_ref, o_ref):
    d = pl.program_id(1)
    hs = hs_ref[...]
    br = br_ref[...]
    z = jnp.concatenate([hs[0], hs[1], br[0], br[1]], axis=1)
    mv = m_ref[...]

    @pl.when(d == 0)
    def _():
        o_ref[...] = jnp.dot(mv, bp_ref[...],
                             preferred_element_type=jnp.float32)

    it = lax.broadcasted_iota(jnp.int32, (400, 128), 1)
    md = jnp.sum(jnp.where(it == d, mv, 0.0), axis=1, keepdims=True)
    o_ref[...] += md * jnp.dot(z.astype(jnp.bfloat16),
                               w_ref[...][0].astype(jnp.bfloat16),
                               preferred_element_type=jnp.float32)


def _mf_mm(hsum2, br2, m, wcat, bp):
    return pl.pallas_call(
        _mf_mm_body,
        grid=(25, ND),
        in_specs=[
            pl.BlockSpec((2, 400, 128), lambda i, d: (0, i, 0)),
            pl.BlockSpec((2, 400, 128), lambda i, d: (0, i, 0)),
            pl.BlockSpec((400, 128), lambda i, d: (i, 0)),
            pl.BlockSpec((1, 512, 256), lambda i, d: (d, 0, 0)),
            pl.BlockSpec((128, 256), lambda i, d: (0, 0)),
        ],
        out_specs=pl.BlockSpec((400, 256), lambda i, d: (i, 0)),
        out_shape=_f32(N, 256),
    )(hsum2, br2, m, wcat, bp)


def _pool_body(h_ref, bi_ref, w1_ref, b1_ref, w2_ref, b2_ref, o_ref, acc):
    i = pl.program_id(0)

    @pl.when(i == 0)
    def _():
        acc[...] = jnp.zeros_like(acc)

    bi = bi_ref[...][0]
    it = lax.broadcasted_iota(jnp.int32, (G, 400), 0)
    mask = (it == bi).astype(jnp.float32)
    acc[...] += jnp.dot(mask, h_ref[...], preferred_element_type=jnp.float32)

    @pl.when(i == 24)
    def _():
        y = acc[...]
        t = jnp.maximum(jnp.dot(y, w1_ref[...],
                                preferred_element_type=jnp.float32)
                        + b1_ref[...], 0.0)
        o_ref[...] = jnp.dot(t, w2_ref[...],
                             preferred_element_type=jnp.float32) + b2_ref[...]


def _pool_head(h, bidx3, w1, b1, w2, b2):
    return pl.pallas_call(
        _pool_body,
        grid=(25,),
        in_specs=[
            pl.BlockSpec((400, 256), lambda i: (i, 0)),
            pl.BlockSpec((1, 1, 400), lambda i: (i, 0, 0)),
            pl.BlockSpec((256, 256), lambda i: (0, 0)),
            pl.BlockSpec((1, 256), lambda i: (0, 0)),
            pl.BlockSpec((256, 128), lambda i: (0, 0)),
            pl.BlockSpec((1, 128), lambda i: (0, 0)),
        ],
        out_specs=pl.BlockSpec((G, 128), lambda i: (0, 0)),
        out_shape=_f32(G, 128),
        scratch_shapes=[pltpu.VMEM((G, 256), jnp.float32)],
    )(h, bidx3, w1, b1, w2, b2)


def kernel(x, edge_index, edge_attr, batch_idx, W_emb, b_emb, rgcn_w,
           rgcn_root, rgcn_b, mf_w, mf_b, mf_root, W1, b1, W2, b2):
    src = edge_index[0]
    dst = edge_index[1]
    srcm = src.reshape(20, 1, 16000)
    edge_attr_t = edge_attr.T
    dstf = dst

    dstm = dst.reshape(20, 1, 16000)
    etm, segm, gix2, src2 = _prep(edge_attr_t, srcm, dstm)
    etf = etm.reshape(E)
    segf = segm.reshape(E)
    gixf = gix2.reshape(2 * E)
    src2f = src2.reshape(2 * E)

    z128 = jnp.zeros((TS, HH), jnp.float32)

    cnt2 = _counts(etf, dstf, z128).reshape(2, NPAD, HH)
    recip, m_onehot = _combine(cnt2)

    w17 = jnp.concatenate([rgcn_w, rgcn_root[:, None]], axis=1)
    wcat = jnp.concatenate([mf_w, mf_root], axis=2)
    bp = jnp.pad(mf_b, ((0, 0), (0, 128 - ND), (0, 0)))
    bh = rgcn_b.reshape(2, 2, 1, 128)

    h = _emb(x, W_emb, b_emb.reshape(1, 256))
    for blk in range(2):
        o_all = _rgcn_mm(h, w17[blk])
        xt = o_all.reshape(2 * XR, HH)
        acc2 = _edge_pass_scaled(xt, gixf, dstf, segf, recip.reshape(N * R),
                                 z128).reshape(2, NPAD, 128)
        br2 = _post_rgcn(acc2, o_all, bh[blk])
        hsum2 = _edge_pass_plain(br2.reshape(2 * N, HH), src2f, dstf,
                                 z128).reshape(2, NPAD, 128)
        h = _mf_mm(hsum2, br2.reshape(2, N, 128), m_onehot, wcat[blk], bp[blk])

    out = _pool_head(h, batch_idx.reshape(25, 1, 400), W1, b1.reshape(1, 256),
                     W2, b2.reshape(1, 128))
    return out
